# baseline (speedup 1.0000x reference)
import functools

import jax
import jax.numpy as jnp
from jax import lax
from jax.experimental import pallas as pl
from jax.experimental.pallas import tpu as pltpu
from jax.experimental.pallas import tpu_sc as plsc

B = 4
N = 10000
E = 320000
D_V = 128
D_E = 16
D_C = 128
D_OUT = 128


NC = 2
NS = 16
NW = NC * NS
WPG = NW // B
ROWS_W = E // WPG
CH = 2000
NCHUNK = ROWS_W // CH
UNROLL = 40

_sc_mesh = plsc.VectorSubcoreMesh(core_axis_name="c", subcore_axis_name="s")


@functools.partial(
    pl.kernel,
    mesh=_sc_mesh,
    compiler_params=pltpu.CompilerParams(use_tc_tiling_on_sc=False),
    out_type=jax.ShapeDtypeStruct((NW, D_E), jnp.float32),
    scratch_types=[
        pltpu.VMEM((CH, D_E), jnp.float32),
        pltpu.VMEM((CH, D_E), jnp.float32),
        pltpu.VMEM((D_E,), jnp.float32),
        pltpu.SemaphoreType.DMA,
        pltpu.SemaphoreType.DMA,
    ],
)
def _edge_partials(edge_hbm, out_hbm, buf0, buf1, acc_v, sem0, sem1):
    c = lax.axis_index("c")
    s = lax.axis_index("s")
    wid = s * NC + c
    g = wid // WPG
    base = (wid % WPG) * ROWS_W

    bufs = (buf0, buf1)
    sems = (sem0, sem1)

    pltpu.async_copy(edge_hbm.at[g, pl.ds(base, CH)], buf0, sem0)
    pltpu.async_copy(edge_hbm.at[g, pl.ds(base + CH, CH)], buf1, sem1)

    nacc = 8
    zero = jnp.zeros((D_E,), jnp.float32)
    accs = (zero,) * nacc

    def chunk_body(k2, accs):
        for half in range(2):
            buf = bufs[half]
            sem = sems[half]
            chunk = 2 * k2 + half
            pltpu.make_async_copy(
                edge_hbm.at[g, pl.ds(base, CH)], buf, sem).wait()

            def inner(i, accs):
                accs = list(accs)
                r0 = i * UNROLL
                for u in range(UNROLL):
                    accs[u % nacc] = accs[u % nacc] + buf[r0 + u]
                return tuple(accs)

            accs = lax.fori_loop(0, CH // UNROLL, inner, accs)

            @pl.when(chunk + 2 < NCHUNK)
            def _():
                pltpu.async_copy(
                    edge_hbm.at[g, pl.ds(base + (chunk + 2) * CH, CH)],
                    buf, sem)
        return accs

    accs = lax.fori_loop(0, NCHUNK // 2, chunk_body, accs)

    total = accs[0]
    for a in accs[1:]:
        total = total + a
    acc_v[...] = total
    pltpu.sync_copy(acc_v, out_hbm.at[wid])


GV = 10
V_C = N // GV


def _vertex_body(v_ref, out_ref, acc_v):
    i = pl.program_id(0)

    @pl.when(i == 0)
    def _init():
        acc_v[...] = jnp.zeros_like(acc_v)

    acc_v[...] += jnp.sum(v_ref[...].reshape(B, V_C // 8, 8, D_V), axis=1)

    @pl.when(i == GV - 1)
    def _final():
        out_ref[...] = jnp.sum(acc_v[...], axis=1) * (1.0 / N)


def _vertex_mean(vertex):
    return pl.pallas_call(
        _vertex_body,
        grid=(GV,),
        in_specs=[pl.BlockSpec((B, V_C, D_V), lambda i: (0, i, 0))],
        out_specs=pl.BlockSpec((B, D_V), lambda i: (0, 0)),
        out_shape=jax.ShapeDtypeStruct((B, D_V), jnp.float32),
        scratch_shapes=[pltpu.VMEM((B, 8, D_V), jnp.float32)],
    )(vertex)


def _combine_body(ctx_ref, vagg_ref, epart_ref, w_ref, b_ref, out_ref):
    ep = epart_ref[...].reshape(B, WPG, D_E)
    e_agg = jnp.sum(ep, axis=1) * (1.0 / E)
    out_ref[...] = (
        jnp.dot(ctx_ref[...], w_ref[0:D_C], preferred_element_type=jnp.float32)
        + jnp.dot(vagg_ref[...], w_ref[D_C:D_C + D_V],
                  preferred_element_type=jnp.float32)
        + jnp.dot(e_agg, w_ref[D_C + D_V:D_C + D_V + D_E],
                  preferred_element_type=jnp.float32)
        + b_ref[...]
    )


def _combine(ctx, v_agg, e_part, W, b_r):
    return pl.pallas_call(
        _combine_body,
        out_shape=jax.ShapeDtypeStruct((B, D_OUT), jnp.float32),
    )(ctx, v_agg, e_part, W, b_r)


def kernel(context, vertex, edge, W, b):
    ctx = context.reshape(B, D_C)
    b_r = b.reshape(1, D_OUT)
    e_part = _edge_partials(edge)
    v_agg = _vertex_mean(vertex)
    out = _combine(ctx, v_agg, e_part, W, b_r)
    return out.reshape(B, 1, D_OUT)


# speedup vs baseline: 15.9581x; 15.9581x over previous
import jax
import jax.numpy as jnp
from jax.experimental import pallas as pl
from jax.experimental.pallas import tpu as pltpu

B = 4
N = 10000
E = 320000
D_V = 128
D_E = 16
D_C = 128
D_OUT = 128

G = 10
V_C = N // G
E_C = E // G


def _body(ctx_ref, v_ref, e_ref, w_ref, b_ref, out_ref, acc_v, acc_e):
    i = pl.program_id(0)

    @pl.when(i == 0)
    def _init():
        acc_v[...] = jnp.zeros_like(acc_v)
        acc_e[...] = jnp.zeros_like(acc_e)

    acc_v[...] += jnp.sum(v_ref[...].reshape(B, V_C // 8, 8, D_V), axis=1)
    acc_e[...] += jnp.sum(e_ref[...].reshape(B, D_E, E_C // 128, 128), axis=2)

    @pl.when(i == pl.num_programs(0) - 1)
    def _final():
        v_agg = jnp.sum(acc_v[...], axis=1) * (1.0 / N)
        e_agg = jnp.sum(acc_e[...], axis=2) * (1.0 / E)
        out = (
            jnp.dot(ctx_ref[...], w_ref[0:D_C], preferred_element_type=jnp.float32)
            + jnp.dot(v_agg, w_ref[D_C:D_C + D_V], preferred_element_type=jnp.float32)
            + jnp.dot(e_agg, w_ref[D_C + D_V:D_C + D_V + D_E],
                      preferred_element_type=jnp.float32)
            + b_ref[...]
        )
        out_ref[...] = out


def kernel(context, vertex, edge, W, b):
    ctx = context.reshape(B, D_C)
    b_r = b.reshape(1, D_OUT)
    edge_t = jnp.transpose(edge, (0, 2, 1))

    out = pl.pallas_call(
        _body,
        grid=(G,),
        in_specs=[
            pl.BlockSpec((B, D_C), lambda i: (0, 0)),
            pl.BlockSpec((B, V_C, D_V), lambda i: (0, i, 0)),
            pl.BlockSpec((B, D_E, E_C), lambda i: (0, 0, i)),
            pl.BlockSpec((D_C + D_V + D_E, D_OUT), lambda i: (0, 0)),
            pl.BlockSpec((1, D_OUT), lambda i: (0, 0)),
        ],
        out_specs=pl.BlockSpec((B, D_OUT), lambda i: (0, 0)),
        out_shape=jax.ShapeDtypeStruct((B, D_OUT), jnp.float32),
        scratch_shapes=[
            pltpu.VMEM((B, 8, D_V), jnp.float32),
            pltpu.VMEM((B, D_E, 128), jnp.float32),
        ],
    )(ctx, vertex, edge_t, W, b_r)
    return out.reshape(B, 1, D_OUT)
